# baseline (speedup 1.0000x reference)
import functools

import jax
import jax.numpy as jnp
from jax import lax
from jax.experimental import pallas as pl
from jax.experimental.pallas import tpu as pltpu
from jax.experimental.pallas import tpu_sc as plsc

VOCAB = 32000
D_MODEL = 2048
B = 4
S = 4096

NUM_TOKENS = B * S
NC = 2
NS = 16
NW = NC * NS
TOK_PER_W = NUM_TOKENS // NW
CHUNK = 16
NCHUNK = TOK_PER_W // CHUNK
NBUF = 3


def _gather_kernel(ids_hbm, table_hbm, out_hbm, idx_v, buf0, buf1, buf2,
                   gsem0, gsem1, gsem2, ssem0, ssem1, ssem2):
  wid = lax.axis_index("s") * NC + lax.axis_index("c")
  base = wid * TOK_PER_W

  bufs = (buf0, buf1, buf2)
  gsems = (gsem0, gsem1, gsem2)
  ssems = (ssem0, ssem1, ssem2)

  pltpu.sync_copy(ids_hbm.at[wid], idx_v)

  def out_slice(i):
    return out_hbm.at[pl.ds(base + i * CHUNK, CHUNK), :]

  def issue_g(i, b):
    pltpu.async_copy(table_hbm.at[idx_v.at[i]], bufs[b], gsems[b])

  def wait_g(i, b):
    pltpu.make_async_copy(table_hbm.at[idx_v.at[i]], bufs[b],
                          gsems[b]).wait()

  def issue_s(i, b):
    pltpu.async_copy(bufs[b], out_slice(i), ssems[b])

  def wait_s(i, b):
    pltpu.make_async_copy(bufs[b], out_slice(i), ssems[b]).wait()

  for b in range(NBUF):
    issue_g(b, b)

  wait_g(0, 0)
  issue_s(0, 0)
  for i in (1, 2):
    b = i % NBUF
    wait_g(i, b)
    issue_s(i, b)
    wait_s(i - 1, (i - 1) % NBUF)
    issue_g(i + 2, (i + 2) % NBUF)

  def body(g, carry):
    for b in range(NBUF):
      i = g * NBUF + b
      wait_g(i, b)
      issue_s(i, b)
      wait_s(i - 1, (b - 1) % NBUF)
      issue_g(i + 2, (b + 2) % NBUF)
    return carry

  lax.fori_loop(1, NCHUNK // NBUF, body, 0)

  for i in (NCHUNK - 2, NCHUNK - 1):
    b = i % NBUF
    wait_g(i, b)
    issue_s(i, b)
    wait_s(i - 1, (i - 1) % NBUF)
  wait_s(NCHUNK - 1, (NCHUNK - 1) % NBUF)


@jax.jit
def _embed(ids_flat, weight):
  mesh = plsc.VectorSubcoreMesh(core_axis_name="c", subcore_axis_name="s")
  k = functools.partial(
      pl.kernel,
      mesh=mesh,
      out_type=jax.ShapeDtypeStruct((NUM_TOKENS, D_MODEL), jnp.float32),
      scratch_types=(
          [pltpu.VMEM((NCHUNK, CHUNK), jnp.int32)]
          + [pltpu.VMEM((CHUNK, D_MODEL), jnp.float32)] * NBUF
          + [pltpu.SemaphoreType.DMA] * (2 * NBUF)
      ),
  )(_gather_kernel)
  ids3 = ids_flat.reshape(NW, NCHUNK, CHUNK)
  return k(ids3, weight)


def kernel(input_ids, attention_mask, labels, weight):
  batch_size, seq_length = input_ids.shape
  position_ids = jnp.arange(seq_length, dtype=jnp.int32)[None, :]
  ids_flat = input_ids.astype(jnp.int32).reshape(-1)
  hidden = _embed(ids_flat, weight).reshape(batch_size, seq_length, D_MODEL)
  return (hidden, attention_mask, position_ids, labels)
